# baseline (speedup 1.0000x reference)
import functools

import jax
import jax.numpy as jnp
from jax import lax
from jax.experimental import pallas as pl
from jax.experimental.pallas import tpu as pltpu
from jax.experimental.pallas import tpu_sc as plsc

VOCAB = 1000000
EMBED_DIM = 64
HIDDEN = 256
BATCH = 16384

CHUNK = 256
CHUNK_SHIFT = CHUNK.bit_length() - 1
N_CHUNKS = VOCAB // CHUNK
TAIL_BASE = N_CHUNKS * CHUNK
PAD_ROWS = 2048
OUT_ROWS = BATCH + PAD_ROWS
NSLOT = 2 * (N_CHUNKS // 32 + 2) if N_CHUNKS // 32 >= 62 else 64
NSLOT = 1 << (NSLOT - 1).bit_length()
PAIR_CAP = BATCH + NSLOT * 16


def _make_sc_gather():
    info = plsc.get_sparse_core_info()
    NC, NS = info.num_cores, info.num_subcores
    NW = NC * NS
    mesh = plsc.VectorSubcoreMesh(core_axis_name="c", subcore_axis_name="s")

    @functools.partial(
        pl.kernel,
        mesh=mesh,
        out_type=jax.ShapeDtypeStruct((OUT_ROWS, 128), jnp.float32),
        scratch_types=[
            pltpu.VMEM((BATCH,), jnp.int32),
            pltpu.VMEM((PAIR_CAP,), jnp.int32),
            pltpu.VMEM((PAIR_CAP,), jnp.int32),
            pltpu.VMEM((NSLOT * 16,), jnp.int32),
            pltpu.VMEM((NSLOT * 16,), jnp.int32),
            pltpu.VMEM((2, 64, CHUNK), jnp.float32),
            pltpu.VMEM((2, 16), jnp.int32),
            pltpu.VMEM((2, 16, 128), jnp.float32),
            pltpu.SemaphoreType.DMA((2,)),
            pltpu.SemaphoreType.DMA((2,)),
        ],
        compiler_params=pltpu.CompilerParams(needs_layout_passes=False),
    )
    def sc_gather(t_hbm, tail_hbm, idx_hbm, out_hbm,
                  idx_v, pidx_v, ppos_v, cnt_v, ptr_v, chunk_v,
                  pos16_v, rows16_v, csem, rsem):
        w = lax.axis_index("s") * NC + lax.axis_index("c")
        lanes = lax.iota(jnp.int32, 16)
        ones = jnp.full((16,), 1, jnp.int32)

        pltpu.sync_copy(idx_hbm, idx_v)

        def zero_body(s, _):
            cnt_v[pl.ds(s * 16, 16)] = jnp.zeros((16,), jnp.int32)
            return _

        lax.fori_loop(0, NSLOT, zero_body, jnp.int32(0))

        def count_body(j, _):
            for u in range(4):
                v = idx_v[pl.ds((j * 4 + u) * 16, 16)]
                k = v >> CHUNK_SHIFT
                mine = (k & (NW - 1)) == w
                slot = ((k - w) >> 5) & (NSLOT - 1)
                plsc.addupdate_scatter(cnt_v, [slot * 16 + lanes], ones,
                                       mask=mine)
            return _

        lax.fori_loop(0, BATCH // 64, count_body, jnp.int32(0))

        def prefix_body(s, base):
            row = cnt_v[pl.ds(s * 16, 16)]
            ptr_v[pl.ds(s * 16, 16)] = base + plsc.cumsum(row) - row
            return base + ((jnp.sum(row) + 15) & ~15)

        lax.fori_loop(0, NSLOT, prefix_body, jnp.int32(0))

        def scatter_body(j, _):
            for u in range(4):
                jj = j * 4 + u
                v = idx_v[pl.ds(jj * 16, 16)]
                k = v >> CHUNK_SHIFT
                mine = (k & (NW - 1)) == w
                slot = ((k - w) >> 5) & (NSLOT - 1)
                ptrs = plsc.load_gather(ptr_v, [slot * 16 + lanes], mask=mine)
                plsc.store_scatter(pidx_v, [ptrs], v, mask=mine)
                plsc.store_scatter(ppos_v, [ptrs], jj * 16 + lanes, mask=mine)
                plsc.addupdate_scatter(ptr_v, [slot * 16 + lanes], ones,
                                       mask=mine)
            return _

        lax.fori_loop(0, BATCH // 64, scatter_body, jnp.int32(0))

        n_main = (N_CHUNKS - 1 - w) // NW + 1

        def stage_start(k, p):
            for r in range(8):
                pltpu.async_copy(
                    t_hbm.at[pl.ds(r * 8, 8), pl.ds(k * CHUNK, CHUNK)],
                    chunk_v.at[p, pl.ds(r * 8, 8), :], csem.at[p])

        def stage_wait(k, p):
            for r in range(8):
                pltpu.make_async_copy(
                    t_hbm.at[pl.ds(r * 8, 8), pl.ds(k * CHUNK, CHUNK)],
                    chunk_v.at[p, pl.ds(r * 8, 8), :], csem.at[p]).wait()

        stage_start(w, 0)

        def process_chunk(s, k, base, gctr):
            row = cnt_v[pl.ds(s * 16, 16)]
            tot = jnp.sum(row)
            p = s & 1
            cbase = k * CHUNK

            def flush_body(g, gc):
                q = gc & 1
                cols = pidx_v[pl.ds(base + g * 16, 16)] - cbase
                posv = ppos_v[pl.ds(base + g * 16, 16)]
                ml = (g * 16 + lanes) < tot
                colsc = jnp.where(ml, cols, 0)
                pad = BATCH + ((g * 16 + lanes + w * 64) & (PAD_ROWS - 1))

                @pl.when(gc >= 2)
                def _():
                    pltpu.make_async_copy(
                        rows16_v.at[q], out_hbm.at[pos16_v.at[q]],
                        rsem.at[q]).wait()

                plsc.store_scatter(pos16_v, [jnp.full((16,), q, jnp.int32),
                                             lanes],
                                   jnp.where(ml, posv, pad))
                qv = jnp.full((16,), q, jnp.int32)
                pv = jnp.full((16,), p, jnp.int32)
                for c in range(EMBED_DIM):
                    cv = jnp.full((16,), c, jnp.int32)
                    vv = plsc.load_gather(chunk_v, [pv, cv, colsc])
                    plsc.store_scatter(rows16_v, [qv, lanes, cv], vv)
                pltpu.async_copy(rows16_v.at[q], out_hbm.at[pos16_v.at[q]],
                                 rsem.at[q])
                return gc + 1

            gctr = lax.fori_loop(0, (tot + 15) >> 4, flush_body, gctr)
            return base + ((tot + 15) & ~15), gctr

        def chunk_body(s, carry):
            base, gctr = carry
            k = w + s * NW
            p = s & 1
            stage_wait(k, p)

            @pl.when(s + 1 < n_main)
            def _():
                stage_start(k + NW, 1 - p)

            return process_chunk(s, k, base, gctr)

        base, gctr = lax.fori_loop(0, n_main, chunk_body,
                                   (jnp.int32(0), jnp.int32(0)))

        def tail_fn():
            s = n_main
            p = s & 1
            pltpu.sync_copy(tail_hbm, chunk_v.at[p, :, pl.ds(0, 128)])
            return process_chunk(s, jnp.int32(N_CHUNKS), base, gctr)

        base, gctr = lax.cond(w == (N_CHUNKS % NW), tail_fn,
                              lambda: (base, gctr))

        @pl.when(gctr >= 2)
        def _():
            q = gctr & 1
            pltpu.make_async_copy(rows16_v.at[q], out_hbm.at[pos16_v.at[q]],
                                  rsem.at[q]).wait()

        @pl.when(gctr >= 1)
        def _():
            q = (gctr - 1) & 1
            pltpu.make_async_copy(rows16_v.at[q], out_hbm.at[pos16_v.at[q]],
                                  rsem.at[q]).wait()

    return sc_gather


def _mlp_body(x_ref, w1_ref, b1_ref, w2_ref, b2_ref, o_ref):
    x = x_ref[:, :EMBED_DIM]
    h = jnp.dot(x, w1_ref[...], preferred_element_type=jnp.float32)
    h = jnp.maximum(h + b1_ref[...], 0.0)
    o = jnp.dot(h, w2_ref[...], preferred_element_type=jnp.float32)
    o_ref[...] = jnp.maximum(o + b2_ref[...], 0.0)


def _mlp(x, W1, b1, W2, b2):
    BB = 2048
    return pl.pallas_call(
        _mlp_body,
        grid=(BATCH // BB,),
        in_specs=[
            pl.BlockSpec((BB, 128), lambda i: (i, 0)),
            pl.BlockSpec((EMBED_DIM, HIDDEN), lambda i: (0, 0)),
            pl.BlockSpec((1, HIDDEN), lambda i: (0, 0)),
            pl.BlockSpec((HIDDEN, EMBED_DIM), lambda i: (0, 0)),
            pl.BlockSpec((1, EMBED_DIM), lambda i: (0, 0)),
        ],
        out_specs=pl.BlockSpec((BB, EMBED_DIM), lambda i: (i, 0)),
        out_shape=jax.ShapeDtypeStruct((BATCH, EMBED_DIM), jnp.float32),
    )(x, W1, b1.reshape(1, HIDDEN), W2, b2.reshape(1, EMBED_DIM))


def kernel(inputs, embedding, W1, b1, W2, b2):
    t = embedding.T
    tail = jnp.pad(t[:, TAIL_BASE:], ((0, 0), (0, 128 - (VOCAB - TAIL_BASE))))
    x = _make_sc_gather()(t, tail, inputs)
    return _mlp(x, W1, b1, W2, b2)


# speedup vs baseline: 1.2369x; 1.2369x over previous
import functools

import jax
import jax.numpy as jnp
from jax import lax
from jax.experimental import pallas as pl
from jax.experimental.pallas import tpu as pltpu
from jax.experimental.pallas import tpu_sc as plsc

VOCAB = 1000000
EMBED_DIM = 64
HIDDEN = 256
BATCH = 16384

CHUNK = 512
CHUNK_SHIFT = CHUNK.bit_length() - 1
N_CHUNKS = VOCAB // CHUNK
TAIL_BASE = N_CHUNKS * CHUNK
PAD_ROWS = 2048
OUT_ROWS = BATCH + PAD_ROWS
NSLOT = 2 * (N_CHUNKS // 32 + 2) if N_CHUNKS // 32 >= 62 else 64
NSLOT = 1 << (NSLOT - 1).bit_length()
PAIR_CAP = BATCH + NSLOT * 16


def _make_sc_gather():
    info = plsc.get_sparse_core_info()
    NC, NS = info.num_cores, info.num_subcores
    NW = NC * NS
    mesh = plsc.VectorSubcoreMesh(core_axis_name="c", subcore_axis_name="s")

    @functools.partial(
        pl.kernel,
        mesh=mesh,
        out_type=jax.ShapeDtypeStruct((OUT_ROWS, 128), jnp.float32),
        scratch_types=[
            pltpu.VMEM((BATCH,), jnp.int32),
            pltpu.VMEM((PAIR_CAP,), jnp.int32),
            pltpu.VMEM((PAIR_CAP,), jnp.int32),
            pltpu.VMEM((NSLOT * 64,), jnp.int32),
            pltpu.VMEM((NSLOT * 64,), jnp.int32),
            pltpu.VMEM((2, 64, CHUNK), jnp.float32),
            pltpu.VMEM((2, 16), jnp.int32),
            pltpu.VMEM((2, 16, 128), jnp.float32),
            pltpu.SemaphoreType.DMA((2,)),
            pltpu.SemaphoreType.DMA((2,)),
        ],
        compiler_params=pltpu.CompilerParams(needs_layout_passes=False),
    )
    def sc_gather(t_hbm, tail_hbm, idx_hbm, out_hbm,
                  idx_v, pidx_v, ppos_v, cnt_v, ptr_v, chunk_v,
                  pos16_v, rows16_v, csem, rsem):
        w = lax.axis_index("s") * NC + lax.axis_index("c")
        lanes = lax.iota(jnp.int32, 16)
        ones = jnp.full((16,), 1, jnp.int32)

        pltpu.sync_copy(idx_hbm, idx_v)

        def zero_body(s, _):
            for u in range(4):
                cnt_v[pl.ds(s * 64 + u * 16, 16)] = jnp.zeros((16,), jnp.int32)
            return _

        lax.fori_loop(0, NSLOT, zero_body, jnp.int32(0))

        def count_body(j, _):
            for u in range(4):
                v = idx_v[pl.ds((j * 4 + u) * 16, 16)]
                k = v >> CHUNK_SHIFT
                mine = (k & (NW - 1)) == w
                slot = ((k - w) >> 5) & (NSLOT - 1)
                plsc.addupdate_scatter(cnt_v, [slot * 64 + u * 16 + lanes],
                                       ones, mask=mine)
            return _

        lax.fori_loop(0, BATCH // 64, count_body, jnp.int32(0))

        def prefix_body(s, base):
            for u in range(4):
                row = cnt_v[pl.ds(s * 64 + u * 16, 16)]
                ptr_v[pl.ds(s * 64 + u * 16, 16)] = (
                    base + plsc.cumsum(row) - row)
                base = base + jnp.sum(row)
            return (base + 15) & ~15

        lax.fori_loop(0, NSLOT, prefix_body, jnp.int32(0))

        def scatter_body(j, _):
            for u in range(4):
                jj = j * 4 + u
                v = idx_v[pl.ds(jj * 16, 16)]
                k = v >> CHUNK_SHIFT
                mine = (k & (NW - 1)) == w
                slot = ((k - w) >> 5) & (NSLOT - 1)
                flat = slot * 64 + u * 16 + lanes
                ptrs = plsc.load_gather(ptr_v, [flat], mask=mine)
                plsc.store_scatter(pidx_v, [ptrs], v, mask=mine)
                plsc.store_scatter(ppos_v, [ptrs], jj * 16 + lanes, mask=mine)
                plsc.addupdate_scatter(ptr_v, [flat], ones, mask=mine)
            return _

        lax.fori_loop(0, BATCH // 64, scatter_body, jnp.int32(0))

        n_main = (N_CHUNKS - 1 - w) // NW + 1

        def stage_start(k, p):
            for r in range(8):
                pltpu.async_copy(
                    t_hbm.at[pl.ds(r * 8, 8), pl.ds(k * CHUNK, CHUNK)],
                    chunk_v.at[p, pl.ds(r * 8, 8), :], csem.at[p])

        def stage_wait(k, p):
            for r in range(8):
                pltpu.make_async_copy(
                    t_hbm.at[pl.ds(r * 8, 8), pl.ds(k * CHUNK, CHUNK)],
                    chunk_v.at[p, pl.ds(r * 8, 8), :], csem.at[p]).wait()

        stage_start(w, 0)

        def process_chunk(s, k, base, gctr):
            tot = jnp.int32(0)
            for u in range(4):
                tot = tot + jnp.sum(cnt_v[pl.ds(s * 64 + u * 16, 16)])
            p = s & 1
            cbase = k * CHUNK

            def flush_body(g, gc):
                q = gc & 1
                cols = pidx_v[pl.ds(base + g * 16, 16)] - cbase
                posv = ppos_v[pl.ds(base + g * 16, 16)]
                ml = (g * 16 + lanes) < tot
                colsc = jnp.where(ml, cols, 0)
                pad = BATCH + ((g * 16 + lanes + w * 64) & (PAD_ROWS - 1))

                @pl.when(gc >= 2)
                def _():
                    pltpu.make_async_copy(
                        rows16_v.at[q], out_hbm.at[pos16_v.at[q]],
                        rsem.at[q]).wait()

                plsc.store_scatter(pos16_v, [jnp.full((16,), q, jnp.int32),
                                             lanes],
                                   jnp.where(ml, posv, pad))
                qv = jnp.full((16,), q, jnp.int32)
                pv = jnp.full((16,), p, jnp.int32)
                for c in range(EMBED_DIM):
                    cv = jnp.full((16,), c, jnp.int32)
                    vv = plsc.load_gather(chunk_v, [pv, cv, colsc])
                    plsc.store_scatter(rows16_v, [qv, lanes, cv], vv)
                pltpu.async_copy(rows16_v.at[q], out_hbm.at[pos16_v.at[q]],
                                 rsem.at[q])
                return gc + 1

            gctr = lax.fori_loop(0, (tot + 15) >> 4, flush_body, gctr)
            return base + ((tot + 15) & ~15), gctr

        def chunk_body(s, carry):
            base, gctr = carry
            k = w + s * NW
            p = s & 1
            stage_wait(k, p)

            @pl.when(s + 1 < n_main)
            def _():
                stage_start(k + NW, 1 - p)

            return process_chunk(s, k, base, gctr)

        base, gctr = lax.fori_loop(0, n_main, chunk_body,
                                   (jnp.int32(0), jnp.int32(0)))

        def tail_fn():
            s = n_main
            p = s & 1
            pltpu.sync_copy(tail_hbm, chunk_v.at[p, :, pl.ds(0, 128)])
            return process_chunk(s, jnp.int32(N_CHUNKS), base, gctr)

        base, gctr = lax.cond(w == (N_CHUNKS % NW), tail_fn,
                              lambda: (base, gctr))

        @pl.when(gctr >= 2)
        def _():
            q = gctr & 1
            pltpu.make_async_copy(rows16_v.at[q], out_hbm.at[pos16_v.at[q]],
                                  rsem.at[q]).wait()

        @pl.when(gctr >= 1)
        def _():
            q = (gctr - 1) & 1
            pltpu.make_async_copy(rows16_v.at[q], out_hbm.at[pos16_v.at[q]],
                                  rsem.at[q]).wait()

    return sc_gather


def _mlp_body(x_ref, w1_ref, b1_ref, w2_ref, b2_ref, o_ref):
    x = x_ref[:, :EMBED_DIM]
    h = jnp.dot(x, w1_ref[...], preferred_element_type=jnp.float32)
    h = jnp.maximum(h + b1_ref[...], 0.0)
    o = jnp.dot(h, w2_ref[...], preferred_element_type=jnp.float32)
    o_ref[...] = jnp.maximum(o + b2_ref[...], 0.0)


def _mlp(x, W1, b1, W2, b2):
    BB = 2048
    return pl.pallas_call(
        _mlp_body,
        grid=(BATCH // BB,),
        in_specs=[
            pl.BlockSpec((BB, 128), lambda i: (i, 0)),
            pl.BlockSpec((EMBED_DIM, HIDDEN), lambda i: (0, 0)),
            pl.BlockSpec((1, HIDDEN), lambda i: (0, 0)),
            pl.BlockSpec((HIDDEN, EMBED_DIM), lambda i: (0, 0)),
            pl.BlockSpec((1, EMBED_DIM), lambda i: (0, 0)),
        ],
        out_specs=pl.BlockSpec((BB, EMBED_DIM), lambda i: (i, 0)),
        out_shape=jax.ShapeDtypeStruct((BATCH, EMBED_DIM), jnp.float32),
    )(x, W1, b1.reshape(1, HIDDEN), W2, b2.reshape(1, EMBED_DIM))


def kernel(inputs, embedding, W1, b1, W2, b2):
    t = embedding.T
    tail = jnp.pad(t[:, TAIL_BASE:], ((0, 0), (0, 128 - (VOCAB - TAIL_BASE))))
    x = _make_sc_gather()(t, tail, inputs)
    return _mlp(x, W1, b1, W2, b2)


# speedup vs baseline: 1.3210x; 1.0680x over previous
import functools

import jax
import jax.numpy as jnp
from jax import lax
from jax.experimental import pallas as pl
from jax.experimental.pallas import tpu as pltpu
from jax.experimental.pallas import tpu_sc as plsc

VOCAB = 1000000
EMBED_DIM = 64
HIDDEN = 256
BATCH = 16384

CHUNK = 512
CHUNK_SHIFT = CHUNK.bit_length() - 1
N_CHUNKS = VOCAB // CHUNK
TAIL_BASE = N_CHUNKS * CHUNK
PAD_ROWS = 2048
OUT_ROWS = BATCH + PAD_ROWS
NSLOT = 2 * (N_CHUNKS // 32 + 2) if N_CHUNKS // 32 >= 62 else 64
NSLOT = 1 << (NSLOT - 1).bit_length()
PAIR_CAP = BATCH + NSLOT * 16


def _make_sc_gather():
    info = plsc.get_sparse_core_info()
    NC, NS = info.num_cores, info.num_subcores
    NW = NC * NS
    mesh = plsc.VectorSubcoreMesh(core_axis_name="c", subcore_axis_name="s")

    @functools.partial(
        pl.kernel,
        mesh=mesh,
        out_type=jax.ShapeDtypeStruct((OUT_ROWS, 128), jnp.float32),
        scratch_types=[
            pltpu.VMEM((BATCH,), jnp.int32),
            pltpu.VMEM((PAIR_CAP,), jnp.int32),
            pltpu.VMEM((PAIR_CAP,), jnp.int32),
            pltpu.VMEM((NSLOT * 64,), jnp.int32),
            pltpu.VMEM((NSLOT * 64,), jnp.int32),
            pltpu.VMEM((2, 64, CHUNK), jnp.float32),
            pltpu.VMEM((2, 16), jnp.int32),
            pltpu.VMEM((2, 16, 128), jnp.float32),
            pltpu.SemaphoreType.DMA((2,)),
            pltpu.SemaphoreType.DMA((2,)),
        ],
        compiler_params=pltpu.CompilerParams(needs_layout_passes=False),
    )
    def sc_gather(t_hbm, tail_hbm, idx_hbm, out_hbm,
                  idx_v, pidx_v, ppos_v, cnt_v, ptr_v, chunk_v,
                  pos16_v, rows16_v, csem, rsem):
        w = lax.axis_index("s") * NC + lax.axis_index("c")
        lanes = lax.iota(jnp.int32, 16)
        ones = jnp.full((16,), 1, jnp.int32)

        pltpu.sync_copy(idx_hbm, idx_v)

        def zero_body(s, _):
            for u in range(4):
                cnt_v[pl.ds(s * 64 + u * 16, 16)] = jnp.zeros((16,), jnp.int32)
            return _

        lax.fori_loop(0, NSLOT, zero_body, jnp.int32(0))

        def count_body(j, _):
            for u in range(4):
                v = idx_v[pl.ds((j * 4 + u) * 16, 16)]
                k = v >> CHUNK_SHIFT
                mine = (k & (NW - 1)) == w
                slot = ((k - w) >> 5) & (NSLOT - 1)
                plsc.addupdate_scatter(cnt_v, [slot * 64 + u * 16 + lanes],
                                       ones, mask=mine)
            return _

        lax.fori_loop(0, BATCH // 64, count_body, jnp.int32(0))

        def prefix_body(s, base):
            for u in range(4):
                row = cnt_v[pl.ds(s * 64 + u * 16, 16)]
                ptr_v[pl.ds(s * 64 + u * 16, 16)] = (
                    base + plsc.cumsum(row) - row)
                base = base + jnp.sum(row)
            return (base + 15) & ~15

        lax.fori_loop(0, NSLOT, prefix_body, jnp.int32(0))

        def scatter_body(j, _):
            for u in range(4):
                jj = j * 4 + u
                v = idx_v[pl.ds(jj * 16, 16)]
                k = v >> CHUNK_SHIFT
                mine = (k & (NW - 1)) == w
                slot = ((k - w) >> 5) & (NSLOT - 1)
                flat = slot * 64 + u * 16 + lanes
                ptrs = plsc.load_gather(ptr_v, [flat], mask=mine)
                plsc.store_scatter(pidx_v, [ptrs], v, mask=mine)
                plsc.store_scatter(ppos_v, [ptrs], jj * 16 + lanes, mask=mine)
                plsc.addupdate_scatter(ptr_v, [flat], ones, mask=mine)
            return _

        lax.fori_loop(0, BATCH // 64, scatter_body, jnp.int32(0))

        n_main = (N_CHUNKS - 1 - w) // NW + 1

        def stage_start(k, p):
            for r in range(8):
                pltpu.async_copy(
                    t_hbm.at[pl.ds(r * 8, 8), pl.ds(k * CHUNK, CHUNK)],
                    chunk_v.at[p, pl.ds(r * 8, 8), :], csem.at[p])

        def stage_wait(k, p):
            for r in range(8):
                pltpu.make_async_copy(
                    t_hbm.at[pl.ds(r * 8, 8), pl.ds(k * CHUNK, CHUNK)],
                    chunk_v.at[p, pl.ds(r * 8, 8), :], csem.at[p]).wait()

        stage_start(w, 0)

        def process_chunk(s, k, base, gctr):
            tot = jnp.int32(0)
            for u in range(4):
                tot = tot + jnp.sum(cnt_v[pl.ds(s * 64 + u * 16, 16)])
            p = s & 1
            cbase = k * CHUNK

            def flush_body(g, gc):
                q = gc & 1
                cols = pidx_v[pl.ds(base + g * 16, 16)] - cbase
                posv = ppos_v[pl.ds(base + g * 16, 16)]
                ml = (g * 16 + lanes) < tot
                colsc = jnp.where(ml, cols, 0)
                pad = BATCH + ((g * 16 + lanes + w * 64) & (PAD_ROWS - 1))

                @pl.when(gc >= 2)
                def _():
                    pltpu.make_async_copy(
                        rows16_v.at[q], out_hbm.at[pos16_v.at[q]],
                        rsem.at[q]).wait()

                plsc.store_scatter(pos16_v, [jnp.full((16,), q, jnp.int32),
                                             lanes],
                                   jnp.where(ml, posv, pad))
                qv = jnp.full((16,), q, jnp.int32)
                pv = jnp.full((16,), p, jnp.int32)
                for c in range(EMBED_DIM):
                    cv = jnp.full((16,), c, jnp.int32)
                    vv = plsc.load_gather(chunk_v, [pv, cv, colsc])
                    plsc.store_scatter(rows16_v, [qv, lanes, cv], vv)
                pltpu.async_copy(rows16_v.at[q], out_hbm.at[pos16_v.at[q]],
                                 rsem.at[q])
                return gc + 1

            gctr = lax.fori_loop(0, (tot + 15) >> 4, flush_body, gctr)
            return base + ((tot + 15) & ~15), gctr

        def chunk_body(s, carry):
            base, gctr = carry
            k = w + s * NW
            p = s & 1

            @pl.when(s + 1 < n_main)
            def _():
                stage_start(k + NW, 1 - p)

            stage_wait(k, p)

            return process_chunk(s, k, base, gctr)

        base, gctr = lax.fori_loop(0, n_main, chunk_body,
                                   (jnp.int32(0), jnp.int32(0)))

        def tail_fn():
            s = n_main
            p = s & 1
            pltpu.sync_copy(tail_hbm, chunk_v.at[p, :, pl.ds(0, 128)])
            return process_chunk(s, jnp.int32(N_CHUNKS), base, gctr)

        base, gctr = lax.cond(w == (N_CHUNKS % NW), tail_fn,
                              lambda: (base, gctr))

        @pl.when(gctr >= 2)
        def _():
            q = gctr & 1
            pltpu.make_async_copy(rows16_v.at[q], out_hbm.at[pos16_v.at[q]],
                                  rsem.at[q]).wait()

        @pl.when(gctr >= 1)
        def _():
            q = (gctr - 1) & 1
            pltpu.make_async_copy(rows16_v.at[q], out_hbm.at[pos16_v.at[q]],
                                  rsem.at[q]).wait()

    return sc_gather


def _mlp_body(x_ref, w1_ref, b1_ref, w2_ref, b2_ref, o_ref):
    x = x_ref[:, :EMBED_DIM]
    h = jnp.dot(x, w1_ref[...], preferred_element_type=jnp.float32)
    h = jnp.maximum(h + b1_ref[...], 0.0)
    o = jnp.dot(h, w2_ref[...], preferred_element_type=jnp.float32)
    o_ref[...] = jnp.maximum(o + b2_ref[...], 0.0)


def _mlp(x, W1, b1, W2, b2):
    BB = 2048
    return pl.pallas_call(
        _mlp_body,
        grid=(BATCH // BB,),
        in_specs=[
            pl.BlockSpec((BB, 128), lambda i: (i, 0)),
            pl.BlockSpec((EMBED_DIM, HIDDEN), lambda i: (0, 0)),
            pl.BlockSpec((1, HIDDEN), lambda i: (0, 0)),
            pl.BlockSpec((HIDDEN, EMBED_DIM), lambda i: (0, 0)),
            pl.BlockSpec((1, EMBED_DIM), lambda i: (0, 0)),
        ],
        out_specs=pl.BlockSpec((BB, EMBED_DIM), lambda i: (i, 0)),
        out_shape=jax.ShapeDtypeStruct((BATCH, EMBED_DIM), jnp.float32),
    )(x, W1, b1.reshape(1, HIDDEN), W2, b2.reshape(1, EMBED_DIM))


def kernel(inputs, embedding, W1, b1, W2, b2):
    t = embedding.T
    tail = jnp.pad(t[:, TAIL_BASE:], ((0, 0), (0, 128 - (VOCAB - TAIL_BASE))))
    x = _make_sc_gather()(t, tail, inputs)
    return _mlp(x, W1, b1, W2, b2)
